# baseline (speedup 1.0000x reference)
import functools

import jax
import jax.numpy as jnp
from jax.experimental import pallas as pl

_NC = 16


def _conv_core(xp, pt, q, w, K, Cin, out_form):
    if out_form:
        a1, b1, w2t, b2, w3t, b3, wagg, bout, smat = w
    else:
        a1, b1, w2t, b2, w3t, b3, wagg, bout = w
    N = xp.shape[0]
    TM = q.shape[0]

    qsq = jnp.sum(q * q, axis=1, keepdims=True)
    psq = jnp.sum(pt * pt, axis=0, keepdims=True)
    dot = jnp.dot(q, pt, preferred_element_type=jnp.float32)
    d = (qsq + psq) - 2.0 * dot

    iota = jax.lax.broadcasted_iota(jnp.int32, (1, N), 1)
    feats = []
    rels = []
    for k in range(K):
        dmin = jnp.min(d, axis=1, keepdims=True)
        cand = jnp.where(d <= dmin, iota, N)
        sel = jnp.min(cand, axis=1, keepdims=True)
        oh = (iota == sel).astype(jnp.float32)
        g = jnp.dot(oh, xp, preferred_element_type=jnp.float32)
        feats.append(g[:, :Cin])
        rels.append(g[:, Cin:] - q)
        if k + 1 < K:
            d = d + oh * 1e30

    msq = jnp.sum(rels[0] * rels[0], axis=1, keepdims=True)
    for r in rels[1:]:
        msq = jnp.maximum(msq, jnp.sum(r * r, axis=1, keepdims=True))
    maxi = jnp.sqrt(msq)
    inv = 1.0 / jnp.where(maxi == 0.0, 1.0, maxi)

    cout = smat.shape[1] if out_form else wagg.shape[1]
    D = cout if out_form else Cin
    jj = jax.lax.broadcasted_iota(jnp.int32, (_NC, _NC * D), 1)
    rr = jax.lax.broadcasted_iota(jnp.int32, (_NC, _NC * D), 0)
    trep = ((jj >= rr * D) & (jj < (rr + 1) * D)).astype(jnp.float32)

    acc = jnp.zeros((TM, cout), jnp.float32)
    for k in range(K):
        p = rels[k] * inv
        h = jnp.maximum(
            jnp.dot(p, a1, preferred_element_type=jnp.float32) + b1, 0.0)
        h = jnp.maximum(
            jnp.dot(h, w2t, preferred_element_type=jnp.float32) + b2, 0.0)
        h = jnp.maximum(
            jnp.dot(h, w3t, preferred_element_type=jnp.float32) + b3, 0.0)
        hrep = jnp.dot(h, trep, preferred_element_type=jnp.float32)
        if out_form:
            u = jnp.dot(feats[k], wagg,
                        preferred_element_type=jnp.float32)
            acc = acc + jnp.dot(u * hrep, smat,
                                preferred_element_type=jnp.float32)
        else:
            frep = jnp.concatenate([feats[k]] * _NC, axis=1)
            acc = acc + jnp.dot(frep * hrep, wagg,
                                preferred_element_type=jnp.float32)
    return jnp.maximum(acc + bout, 0.0)


def _prep_weights(p, bnp, K, Cin):
    a1 = p["l1w"].reshape(2 * _NC, 3, _NC).sum(-1).T
    cflat = p["centers"].reshape(1, 3 * _NC)
    b1p = p["l1b"][None] - cflat @ p["l1w"].T
    scale = bnp["g"] / (K * jnp.sqrt(1.0 + 1e-5))
    cout = p["weight"].shape[2]
    out_form = cout < Cin
    w = [a1, b1p, p["l2w"].T, p["l2b"][None], p["l3w"].T, p["l3b"][None]]
    if out_form:
        wagg = (p["weight"] * scale[None, None, :]).reshape(Cin, _NC * cout)
        smat = (jnp.arange(_NC * cout)[:, None] % cout
                == jnp.arange(cout)[None, :]).astype(jnp.float32)
        w += [wagg, bnp["b"][None], smat]
    else:
        wagg = jnp.transpose(p["weight"], (1, 0, 2)).reshape(_NC * Cin, cout)
        w += [wagg * scale[None, :], bnp["b"][None]]
    return w, out_form


def _layer_body(xp_ref, pt_ref, q_ref, *rest, K, Cin, out_form, nw, has_fc):
    wrefs, rest = rest[:nw], rest[nw:]
    if has_fc:
        fct_ref, fcb_ref, o_ref = rest
    else:
        (o_ref,) = rest
    w = [r[...] for r in wrefs]
    r = _conv_core(xp_ref[0], pt_ref[0], q_ref[0], w, K, Cin, out_form)
    if has_fc:
        r = jnp.dot(r, fct_ref[...],
                    preferred_element_type=jnp.float32) + fcb_ref[...]
    o_ref[0] = r


def _ptconv(p, bnp, x, points, K, nxt, fc=None):
    B, N, Cin = x.shape
    M = nxt.shape[1]
    TM = min(M, 1024 if M >= 8192 else 256)

    w, out_form = _prep_weights(p, bnp, K, Cin)
    cout = p["weight"].shape[2]

    xp = jnp.concatenate([x, points], axis=2)
    ptt = jnp.transpose(points, (0, 2, 1))

    args = [xp, ptt, nxt] + w
    wspecs = [pl.BlockSpec(a.shape, lambda b, m: (0, 0)) for a in w]
    cout_eff = cout
    if fc is not None:
        fcw, fcb = fc
        args += [fcw.T, fcb[None]]
        cout_eff = fcw.shape[0]
        wspecs += [
            pl.BlockSpec((cout, cout_eff), lambda b, m: (0, 0)),
            pl.BlockSpec((1, cout_eff), lambda b, m: (0, 0)),
        ]

    out = pl.pallas_call(
        functools.partial(_layer_body, K=K, Cin=Cin, out_form=out_form,
                          nw=len(w), has_fc=fc is not None),
        grid=(B, M // TM),
        in_specs=[
            pl.BlockSpec((1, N, Cin + 3), lambda b, m: (b, 0, 0)),
            pl.BlockSpec((1, 3, N), lambda b, m: (b, 0, 0)),
            pl.BlockSpec((1, TM, 3), lambda b, m: (b, m, 0)),
        ] + wspecs,
        out_specs=pl.BlockSpec((1, TM, cout_eff), lambda b, m: (b, m, 0)),
        out_shape=jax.ShapeDtypeStruct((B, M, cout_eff), jnp.float32),
    )(*args)
    return out


_MID = (
    ("cv3", "bn3", 16, 1024, 256, 48, None),
    ("cv4", "bn4", 8, 256, 64, 48, None),
    ("cv5", "bn5", 8, 64, 16, 96, None),
    ("cv6", "bn6", 4, 16, 8, 96, None),
    ("cv5d", "bn5d", 4, 8, 16, 96, "x5"),
    ("cv4d", "bn4d", 4, 16, 64, 192, "x4"),
    ("cv3d", "bn3d", 4, 64, 256, 192, "x3"),
    ("cv2d", "bn2d", 8, 256, 1024, 96, "x2"),
)


def _mid_body(x2_ref, p2_ref, p2t_ref, *rest, cfg):
    o_ref = rest[-1]
    wrefs = rest[:-1]
    x2 = x2_ref[0]
    p2 = p2_ref[0]
    p2t = p2t_ref[0]

    saved = {"x2": x2}
    cur = x2
    wi = 0
    for name, K, N, M, Cin, out_form, nw, skip in cfg:
        w = [r[...] for r in wrefs[wi:wi + nw]]
        wi += nw
        xp = jnp.concatenate([cur, p2[:N]], axis=1)
        y = _conv_core(xp, p2t[:, :N], p2[:M], w, K, Cin, out_form)
        if skip is not None:
            y = jnp.concatenate([y, saved[skip]], axis=1)
        if name in ("x3", "x4", "x5"):
            saved[name] = y
        cur = y
    o_ref[0] = cur


def _mid_stack(params, x2, pts2):
    B = x2.shape[0]
    p2t = jnp.transpose(pts2, (0, 2, 1))
    wflat = []
    cfg = []
    for cv, bn, K, N, M, Cin, skip in _MID:
        w, out_form = _prep_weights(params[cv], params[bn], K, Cin)
        wflat += w
        save = {"cv3": "x3", "cv4": "x4", "cv5": "x5"}.get(cv)
        cfg.append((save, K, N, M, Cin, out_form, len(w), skip))
    wspecs = [pl.BlockSpec(a.shape, lambda b: (0, 0)) for a in wflat]
    return pl.pallas_call(
        functools.partial(_mid_body, cfg=tuple(cfg)),
        grid=(B,),
        in_specs=[
            pl.BlockSpec((1, 1024, 48), lambda b: (b, 0, 0)),
            pl.BlockSpec((1, 1024, 3), lambda b: (b, 0, 0)),
            pl.BlockSpec((1, 3, 1024), lambda b: (b, 0, 0)),
        ] + wspecs,
        out_specs=pl.BlockSpec((1, 1024, 96), lambda b: (b, 0, 0)),
        out_shape=jax.ShapeDtypeStruct((B, 1024, 96), jnp.float32),
    )(x2, pts2, p2t, *wflat)


def kernel(x, input_pts, params):
    P = params
    pts2 = input_pts[:, :1024]
    x2 = _ptconv(P["cv2"], P["bn2"], x, input_pts, 16, pts2)
    x2d = _mid_stack(P, x2, pts2)
    return _ptconv(P["cv1d"], P["bn1d"], x2d, pts2, 8, input_pts,
                   fc=(P["fc_w"], P["fc_b"]))


# speedup vs baseline: 1.0662x; 1.0662x over previous
import functools

import jax
import jax.numpy as jnp
from jax.experimental import pallas as pl

_NC = 16


def _conv_core(xp, pt, q, w, K, Cin, out_form):
    if out_form:
        a1, b1, w2t, b2, w3t, b3, wagg, bout, smat = w
    else:
        a1, b1, w2t, b2, w3t, b3, wagg, bout = w
    N = xp.shape[0]
    TM = q.shape[0]

    qsq = jnp.sum(q * q, axis=1, keepdims=True)
    psq = jnp.sum(pt * pt, axis=0, keepdims=True)
    dot = jnp.dot(q, pt, preferred_element_type=jnp.float32)
    d = (qsq + psq) - 2.0 * dot

    iota = jax.lax.broadcasted_iota(jnp.int32, (1, N), 1)
    feats = []
    rels = []
    for k in range(K):
        dmin = jnp.min(d, axis=1, keepdims=True)
        cand = jnp.where(d <= dmin, iota, N)
        sel = jnp.min(cand, axis=1, keepdims=True)
        hit = iota == sel
        oh = hit.astype(jnp.float32)
        g = jnp.dot(oh, xp, preferred_element_type=jnp.float32)
        feats.append(g[:, :Cin])
        rels.append(g[:, Cin:] - q)
        if k + 1 < K:
            d = jnp.where(hit, 1e30, d)

    msq = jnp.sum(rels[0] * rels[0], axis=1, keepdims=True)
    for r in rels[1:]:
        msq = jnp.maximum(msq, jnp.sum(r * r, axis=1, keepdims=True))
    maxi = jnp.sqrt(msq)
    inv = 1.0 / jnp.where(maxi == 0.0, 1.0, maxi)

    cout = smat.shape[1] if out_form else wagg.shape[1]
    D = cout if out_form else Cin
    jj = jax.lax.broadcasted_iota(jnp.int32, (_NC, _NC * D), 1)
    rr = jax.lax.broadcasted_iota(jnp.int32, (_NC, _NC * D), 0)
    trep = ((jj >= rr * D) & (jj < (rr + 1) * D)).astype(jnp.float32)

    acc = jnp.zeros((TM, cout), jnp.float32)
    for k in range(K):
        p = rels[k] * inv
        h = jnp.maximum(
            jnp.dot(p, a1, preferred_element_type=jnp.float32) + b1, 0.0)
        h = jnp.maximum(
            jnp.dot(h, w2t, preferred_element_type=jnp.float32) + b2, 0.0)
        h = jnp.maximum(
            jnp.dot(h, w3t, preferred_element_type=jnp.float32) + b3, 0.0)
        hrep = jnp.dot(h, trep, preferred_element_type=jnp.float32)
        if out_form:
            u = jnp.dot(feats[k], wagg,
                        preferred_element_type=jnp.float32)
            acc = acc + jnp.dot(u * hrep, smat,
                                preferred_element_type=jnp.float32)
        else:
            frep = jnp.concatenate([feats[k]] * _NC, axis=1)
            acc = acc + jnp.dot(frep * hrep, wagg,
                                preferred_element_type=jnp.float32)
    return jnp.maximum(acc + bout, 0.0)


def _prep_weights(p, bnp, K, Cin):
    a1 = p["l1w"].reshape(2 * _NC, 3, _NC).sum(-1).T
    cflat = p["centers"].reshape(1, 3 * _NC)
    b1p = p["l1b"][None] - cflat @ p["l1w"].T
    scale = bnp["g"] / (K * jnp.sqrt(1.0 + 1e-5))
    cout = p["weight"].shape[2]
    out_form = cout < Cin
    w = [a1, b1p, p["l2w"].T, p["l2b"][None], p["l3w"].T, p["l3b"][None]]
    if out_form:
        wagg = (p["weight"] * scale[None, None, :]).reshape(Cin, _NC * cout)
        smat = (jnp.arange(_NC * cout)[:, None] % cout
                == jnp.arange(cout)[None, :]).astype(jnp.float32)
        w += [wagg, bnp["b"][None], smat]
    else:
        wagg = jnp.transpose(p["weight"], (1, 0, 2)).reshape(_NC * Cin, cout)
        w += [wagg * scale[None, :], bnp["b"][None]]
    return w, out_form


def _layer_body(xp_ref, pt_ref, q_ref, *rest, K, Cin, out_form, nw, has_fc):
    wrefs, rest = rest[:nw], rest[nw:]
    if has_fc:
        fct_ref, fcb_ref, o_ref = rest
    else:
        (o_ref,) = rest
    w = [r[...] for r in wrefs]
    r = _conv_core(xp_ref[0], pt_ref[0], q_ref[0], w, K, Cin, out_form)
    if has_fc:
        r = jnp.dot(r, fct_ref[...],
                    preferred_element_type=jnp.float32) + fcb_ref[...]
    o_ref[0] = r


def _ptconv(p, bnp, x, points, K, nxt, fc=None):
    B, N, Cin = x.shape
    M = nxt.shape[1]
    TM = min(M, 1024 if M >= 8192 else 256)

    w, out_form = _prep_weights(p, bnp, K, Cin)
    cout = p["weight"].shape[2]

    xp = jnp.concatenate([x, points], axis=2)
    ptt = jnp.transpose(points, (0, 2, 1))

    args = [xp, ptt, nxt] + w
    wspecs = [pl.BlockSpec(a.shape, lambda b, m: (0, 0)) for a in w]
    cout_eff = cout
    if fc is not None:
        fcw, fcb = fc
        args += [fcw.T, fcb[None]]
        cout_eff = fcw.shape[0]
        wspecs += [
            pl.BlockSpec((cout, cout_eff), lambda b, m: (0, 0)),
            pl.BlockSpec((1, cout_eff), lambda b, m: (0, 0)),
        ]

    out = pl.pallas_call(
        functools.partial(_layer_body, K=K, Cin=Cin, out_form=out_form,
                          nw=len(w), has_fc=fc is not None),
        grid=(B, M // TM),
        in_specs=[
            pl.BlockSpec((1, N, Cin + 3), lambda b, m: (b, 0, 0)),
            pl.BlockSpec((1, 3, N), lambda b, m: (b, 0, 0)),
            pl.BlockSpec((1, TM, 3), lambda b, m: (b, m, 0)),
        ] + wspecs,
        out_specs=pl.BlockSpec((1, TM, cout_eff), lambda b, m: (b, m, 0)),
        out_shape=jax.ShapeDtypeStruct((B, M, cout_eff), jnp.float32),
    )(*args)
    return out


_MID = (
    ("cv3", "bn3", 16, 1024, 256, 48, None),
    ("cv4", "bn4", 8, 256, 64, 48, None),
    ("cv5", "bn5", 8, 64, 16, 96, None),
    ("cv6", "bn6", 4, 16, 8, 96, None),
    ("cv5d", "bn5d", 4, 8, 16, 96, "x5"),
    ("cv4d", "bn4d", 4, 16, 64, 192, "x4"),
    ("cv3d", "bn3d", 4, 64, 256, 192, "x3"),
    ("cv2d", "bn2d", 8, 256, 1024, 96, "x2"),
)


def _mid_body(x2_ref, p2_ref, p2t_ref, *rest, cfg):
    o_ref = rest[-1]
    wrefs = rest[:-1]
    x2 = x2_ref[0]
    p2 = p2_ref[0]
    p2t = p2t_ref[0]

    saved = {"x2": x2}
    cur = x2
    wi = 0
    for name, K, N, M, Cin, out_form, nw, skip in cfg:
        w = [r[...] for r in wrefs[wi:wi + nw]]
        wi += nw
        xp = jnp.concatenate([cur, p2[:N]], axis=1)
        y = _conv_core(xp, p2t[:, :N], p2[:M], w, K, Cin, out_form)
        if skip is not None:
            y = jnp.concatenate([y, saved[skip]], axis=1)
        if name in ("x3", "x4", "x5"):
            saved[name] = y
        cur = y
    o_ref[0] = cur


def _mid_stack(params, x2, pts2):
    B = x2.shape[0]
    p2t = jnp.transpose(pts2, (0, 2, 1))
    wflat = []
    cfg = []
    for cv, bn, K, N, M, Cin, skip in _MID:
        w, out_form = _prep_weights(params[cv], params[bn], K, Cin)
        wflat += w
        save = {"cv3": "x3", "cv4": "x4", "cv5": "x5"}.get(cv)
        cfg.append((save, K, N, M, Cin, out_form, len(w), skip))
    wspecs = [pl.BlockSpec(a.shape, lambda b: (0, 0)) for a in wflat]
    return pl.pallas_call(
        functools.partial(_mid_body, cfg=tuple(cfg)),
        grid=(B,),
        in_specs=[
            pl.BlockSpec((1, 1024, 48), lambda b: (b, 0, 0)),
            pl.BlockSpec((1, 1024, 3), lambda b: (b, 0, 0)),
            pl.BlockSpec((1, 3, 1024), lambda b: (b, 0, 0)),
        ] + wspecs,
        out_specs=pl.BlockSpec((1, 1024, 96), lambda b: (b, 0, 0)),
        out_shape=jax.ShapeDtypeStruct((B, 1024, 96), jnp.float32),
    )(x2, pts2, p2t, *wflat)


def kernel(x, input_pts, params):
    P = params
    pts2 = input_pts[:, :1024]
    x2 = _ptconv(P["cv2"], P["bn2"], x, input_pts, 16, pts2)
    x2d = _mid_stack(P, x2, pts2)
    return _ptconv(P["cv1d"], P["bn1d"], x2d, pts2, 8, input_pts,
                   fc=(P["fc_w"], P["fc_b"]))
